# Initial kernel scaffold; baseline (speedup 1.0000x reference)
import jax
import jax.numpy as jnp
from jax.experimental import pallas as pl


def kernel(pos, pe_weight):
    raise NotImplementedError("write your pallas kernel here")


# speedup vs baseline: 2.3689x; 2.3689x over previous
import functools

import jax
import jax.numpy as jnp
from jax import lax
from jax.experimental import pallas as pl
from jax.experimental.pallas import tpu as pltpu
from jax.experimental.pallas import tpu_sc as plsc

_info = plsc.get_sparse_core_info()
_NC, _NS = _info.num_cores, _info.num_subcores
_NW = _NC * _NS

_K = 32


def _gather_kernel(table_hbm, idx_hbm, out_hbm,
                   idx_v, buf0, buf1, gsem0, gsem1, osem0, osem1):
    n_idx = idx_hbm.shape[0]
    b_per_w = n_idx // _NW
    n_chunks = b_per_w // _K
    wid = lax.axis_index("s") * _NC + lax.axis_index("c")
    base = wid * b_per_w

    pltpu.sync_copy(idx_hbm.at[pl.ds(base, b_per_w)], idx_v)

    bufs = (buf0, buf1)
    gsems = (gsem0, gsem1)
    osems = (osem0, osem1)

    def start_gather(g, b):
        pltpu.async_copy(table_hbm.at[idx_v.at[pl.ds(g * _K, _K)]],
                         bufs[b], gsems[b])

    def wait_gather(b):
        pltpu.make_async_copy(table_hbm.at[idx_v.at[pl.ds(0, _K)]],
                              bufs[b], gsems[b]).wait()

    def start_out(g, b):
        pltpu.async_copy(bufs[b], out_hbm.at[pl.ds(base + g * _K, _K)],
                         osems[b])

    def wait_out(b):
        pltpu.make_async_copy(bufs[b], out_hbm.at[pl.ds(base, _K)],
                              osems[b]).wait()

    start_gather(0, 0)
    start_gather(1, 1)

    def group(go, carry):
        for b in range(2):
            g = go * 2 + b
            wait_gather(b)
            start_out(g, b)
            wait_out(b)
            start_gather(g + 2, b)
        return carry

    lax.fori_loop(0, n_chunks // 2 - 1, group, 0, unroll=False)

    for b in range(2):
        g = n_chunks - 2 + b
        wait_gather(b)
        start_out(g, b)
    for b in range(2):
        wait_out(b)


@functools.partial(jax.jit, static_argnames=())
def kernel(pos, pe_weight):
    batch, seq = pos.shape
    dim = pe_weight.shape[1]
    n_idx = batch * seq
    flat_pos = pos.reshape(n_idx).astype(jnp.int32)
    b_per_w = n_idx // _NW

    mesh = plsc.VectorSubcoreMesh(core_axis_name="c", subcore_axis_name="s")
    run = pl.kernel(
        _gather_kernel,
        out_type=jax.ShapeDtypeStruct((n_idx, dim), jnp.float32),
        mesh=mesh,
        scratch_types=[
            pltpu.VMEM((b_per_w,), jnp.int32),
            pltpu.VMEM((_K, dim), jnp.float32),
            pltpu.VMEM((_K, dim), jnp.float32),
            pltpu.SemaphoreType.DMA,
            pltpu.SemaphoreType.DMA,
            pltpu.SemaphoreType.DMA,
            pltpu.SemaphoreType.DMA,
        ],
    )
    out = run(pe_weight, flat_pos)
    return out.reshape(batch, seq, dim)
